# baseline (speedup 1.0000x reference)
import functools

import jax
import jax.numpy as jnp
from jax import lax
from jax.experimental import pallas as pl
from jax.experimental.pallas import tpu as pltpu
from jax.experimental.pallas import tpu_sc as plsc

N_VECTORS = 8192
VECTOR_DIM = 32
TOTAL = 16384
BETA = 0.25

_R = 1024
_C = 8192
_RB = TOTAL // _R
_CB = N_VECTORS // _C

_NC = 2
_NS = 16
_NW = _NC * _NS
_BPW = (TOTAL // 2) // _NW


def _argmin_body(x_ref, cb_ref, idx_ref, loss_ref, colf_s):
    i = pl.program_id(0)

    @pl.when(i == 0)
    def _():
        colf_s[...] = lax.broadcasted_iota(
            jnp.int32, (8, _C), 1).astype(jnp.float32)

    x = x_ref[...]
    e = cb_ref[...]
    xnorm = jnp.sum(x * x, axis=1, keepdims=True)
    cnorm = jnp.sum(e * e, axis=0, keepdims=True)
    sim = jnp.dot(x, e, preferred_element_type=jnp.float32)
    d = (xnorm + cnorm) - 2.0 * sim
    minv = jnp.min(d, axis=1, keepdims=True)
    colf = colf_s[0:1, :]
    minf = jnp.min(jnp.where(d == minv, colf, jnp.float32(2.0**30)),
                   axis=1, keepdims=True)
    idx_ref[...] = minf.astype(jnp.int32)[:, 0]

    @pl.when(i == 0)
    def _():
        loss_ref[0, 0] = jnp.float32(0.0)

    loss_ref[0, 0] += jnp.sum(minv)


_HALF = TOTAL // 2
_RBH = _HALF // _R

_argmin_call = pl.pallas_call(
    _argmin_body,
    grid=(_RBH,),
    in_specs=[
        pl.BlockSpec((_R, VECTOR_DIM), lambda i: (i, 0)),
        pl.BlockSpec((VECTOR_DIM, _C), lambda i: (0, 0)),
    ],
    out_specs=[
        pl.BlockSpec((_R,), lambda i: (i,)),
        pl.BlockSpec(memory_space=pltpu.SMEM),
    ],
    out_shape=[
        jax.ShapeDtypeStruct((_HALF,), jnp.int32),
        jax.ShapeDtypeStruct((1, 1), jnp.float32),
    ],
    scratch_shapes=[
        pltpu.VMEM((8, _C), jnp.float32),
    ],
    compiler_params=pltpu.CompilerParams(
        dimension_semantics=("arbitrary",)),
)


@functools.cache
def _gather_rows():
    @functools.partial(
        pl.kernel,
        out_type=jax.ShapeDtypeStruct((_HALF, VECTOR_DIM), jnp.float32),
        mesh=plsc.VectorSubcoreMesh(
            core_axis_name="c", subcore_axis_name="s",
            num_cores=_NC, num_subcores=_NS),
        scratch_types=[
            pltpu.VMEM((_BPW,), jnp.int32),
            pltpu.VMEM((_BPW, VECTOR_DIM), jnp.float32),
            pltpu.SemaphoreType.DMA,
        ],
        compiler_params=pltpu.CompilerParams(use_tc_tiling_on_sc=False),
    )
    def gather(table_hbm, idx_hbm, out_hbm, idx_v, rows_v, sem):
        wid = lax.axis_index("s") * _NC + lax.axis_index("c")
        base = wid * _BPW
        pltpu.sync_copy(idx_hbm.at[pl.ds(base, _BPW)], idx_v)
        pltpu.async_copy(table_hbm.at[idx_v], rows_v, sem).wait()
        pltpu.sync_copy(rows_v, out_hbm.at[pl.ds(base, _BPW)])

    return gather


def kernel(inputs, quantized_vectors):
    x = inputs.reshape(TOTAL, VECTOR_DIM)
    table = quantized_vectors.T
    gather = _gather_rows()
    idx_a, loss_a = _argmin_call(x[:_HALF], quantized_vectors)
    q_a = gather(table, idx_a)
    idx_b, loss_b = _argmin_call(x[_HALF:], quantized_vectors)
    q_b = gather(table, idx_b)
    quantized = jnp.concatenate([q_a, q_b], axis=0)
    vq_loss = (loss_a[0, 0] + loss_b[0, 0]) * jnp.float32(
        (1.0 + BETA) / (TOTAL * VECTOR_DIM))
    return quantized.reshape(inputs.shape), vq_loss


# speedup vs baseline: 1.0470x; 1.0470x over previous
import functools

import jax
import jax.numpy as jnp
from jax import lax
from jax.experimental import pallas as pl
from jax.experimental.pallas import tpu as pltpu
from jax.experimental.pallas import tpu_sc as plsc

N_VECTORS = 8192
VECTOR_DIM = 32
TOTAL = 16384
BETA = 0.25

_R = 1024
_C = 8192
_RB = TOTAL // _R
_CB = N_VECTORS // _C

_NC = 2
_NS = 16
_NW = _NC * _NS
_BPW = TOTAL // _NW


def _argmin_body(x_ref, cb_ref, idx_ref, loss_ref, colf_s):
    i = pl.program_id(0)

    @pl.when(i == 0)
    def _():
        colf_s[...] = lax.broadcasted_iota(
            jnp.int32, (8, _C), 1).astype(jnp.float32)

    x = x_ref[...]
    e = cb_ref[...]
    xnorm = jnp.sum(x * x, axis=1, keepdims=True)
    cnorm = jnp.sum(e * e, axis=0, keepdims=True)
    sim = jnp.dot(x, e, preferred_element_type=jnp.float32)
    d = (xnorm + cnorm) - 2.0 * sim
    minv = jnp.min(d, axis=1, keepdims=True)
    colf = colf_s[0:1, :]
    minf = jnp.min(jnp.where(d == minv, colf, jnp.float32(2.0**30)),
                   axis=1, keepdims=True)
    idx_ref[...] = minf.astype(jnp.int32)[:, 0]

    @pl.when(i == 0)
    def _():
        loss_ref[0, 0] = jnp.float32(0.0)

    loss_ref[0, 0] += jnp.sum(minv)

    @pl.when(i == _RB - 1)
    def _():
        loss_ref[0, 0] = loss_ref[0, 0] * jnp.float32(
            (1.0 + BETA) / (TOTAL * VECTOR_DIM))


_argmin_call = pl.pallas_call(
    _argmin_body,
    grid=(_RB,),
    in_specs=[
        pl.BlockSpec((_R, VECTOR_DIM), lambda i: (i, 0)),
        pl.BlockSpec((VECTOR_DIM, _C), lambda i: (0, 0)),
    ],
    out_specs=[
        pl.BlockSpec((_R,), lambda i: (i,)),
        pl.BlockSpec(memory_space=pltpu.SMEM),
    ],
    out_shape=[
        jax.ShapeDtypeStruct((TOTAL,), jnp.int32),
        jax.ShapeDtypeStruct((1, 1), jnp.float32),
    ],
    scratch_shapes=[
        pltpu.VMEM((8, _C), jnp.float32),
    ],
    compiler_params=pltpu.CompilerParams(
        dimension_semantics=("arbitrary",)),
)


@functools.cache
def _gather_rows():
    @functools.partial(
        pl.kernel,
        out_type=jax.ShapeDtypeStruct((TOTAL, 128), jnp.float32),
        mesh=plsc.VectorSubcoreMesh(
            core_axis_name="c", subcore_axis_name="s",
            num_cores=_NC, num_subcores=_NS),
        scratch_types=[
            pltpu.VMEM((_BPW,), jnp.int32),
            pltpu.VMEM((_BPW, 128), jnp.float32),
            pltpu.SemaphoreType.DMA,
        ],
    )
    def gather(table_hbm, idx_hbm, out_hbm, idx_v, rows_v, sem):
        wid = lax.axis_index("s") * _NC + lax.axis_index("c")
        base = wid * _BPW
        pltpu.sync_copy(idx_hbm.at[pl.ds(base, _BPW)], idx_v)
        pltpu.async_copy(table_hbm.at[idx_v], rows_v, sem).wait()
        pltpu.sync_copy(rows_v, out_hbm.at[pl.ds(base, _BPW)])

    return gather


def kernel(inputs, quantized_vectors):
    x = inputs.reshape(TOTAL, VECTOR_DIM)
    idx2d, loss = _argmin_call(x, quantized_vectors)
    table = jnp.pad(quantized_vectors.T, ((0, 0), (0, 96)))
    quantized = _gather_rows()(table, idx2d)[:, :VECTOR_DIM]
    return quantized.reshape(inputs.shape), loss[0, 0]
